# baseline (speedup 1.0000x reference)
import functools

import jax
import jax.numpy as jnp
from jax import lax
from jax.experimental import pallas as pl
from jax.experimental.pallas import tpu as pltpu
from jax.experimental.pallas import tpu_sc as plsc

D_MODEL = 64
SCALE = float(D_MODEL) ** 0.5
NC = 2
NS = 16
NW = NC * NS
CHUNK = 128
NBUF = 2


@functools.lru_cache(maxsize=None)
def _build(batch, vocab):
    assert batch % (NW * CHUNK) == 0
    per_w = batch // NW
    n_chunks = per_w // CHUNK
    mesh = plsc.VectorSubcoreMesh(core_axis_name="c", subcore_axis_name="s")

    @functools.partial(
        pl.kernel,
        out_type=jax.ShapeDtypeStruct((batch, D_MODEL), jnp.float32),
        mesh=mesh,
        scratch_types=[
            pltpu.VMEM((n_chunks, CHUNK), jnp.int32),
            pltpu.VMEM((NBUF, CHUNK, D_MODEL), jnp.float32),
            pltpu.SemaphoreType.DMA,
            pltpu.SemaphoreType.DMA,
        ],
        compiler_params=pltpu.CompilerParams(use_tc_tiling_on_sc=False),
    )
    def emb(idx_hbm, table_hbm, out_hbm, idx_v, rows_v, sem0, sem1):
        wid = lax.axis_index("s") * NC + lax.axis_index("c")
        base = wid * per_w
        sems = (sem0, sem1)

        pltpu.sync_copy(idx_hbm.at[wid], idx_v)

        def start(c, b):
            pltpu.async_copy(table_hbm.at[idx_v.at[c]], rows_v.at[b], sems[b])

        def wait(b):
            pltpu.make_async_copy(
                table_hbm.at[pl.ds(0, CHUNK)], rows_v.at[b], sems[b]
            ).wait()

        def process(c, b):
            def sbody(i, _):
                for j in range(D_MODEL // 16):
                    sl = pl.ds(j * 16, 16)
                    rows_v[b, i, sl] = rows_v[b, i, sl] * SCALE
                return 0

            lax.fori_loop(0, CHUNK, sbody, 0)
            pltpu.sync_copy(
                rows_v.at[b], out_hbm.at[pl.ds(base + c * CHUNK, CHUNK)]
            )

        for b in range(NBUF):
            start(b, b)

        def gbody(g, _):
            for b in range(NBUF):
                c = g * NBUF + b
                wait(b)
                process(c, b)

                @pl.when(c + NBUF < n_chunks)
                def _():
                    start(c + NBUF, b)

            return 0

        lax.fori_loop(0, n_chunks // NBUF, gbody, 0)

    return emb


def kernel(x, table):
    batch = x.size
    idx = x.reshape(NW, batch // NW // CHUNK, CHUNK).astype(jnp.int32)
    out = _build(batch, table.shape[0])(idx, table)
    return out.reshape(*x.shape, D_MODEL)
